# baseline (speedup 1.0000x reference)
import functools

import jax
import jax.numpy as jnp
from jax import lax
from jax.experimental import pallas as pl
from jax.experimental.pallas import tpu as pltpu
from jax.experimental.pallas import tpu_sc as plsc

B = 4096
N_SPARSE = 26
N_DENSE = 13
N_COLS = N_SPARSE + N_DENSE
VOCAB = 100000


@functools.cache
def _build():
    info = plsc.get_sparse_core_info()
    NC, NS, L = info.num_cores, info.num_subcores, info.num_lanes
    NW = NC * NS
    RPW = B // NW
    NG = RPW // L
    mesh = plsc.VectorSubcoreMesh(core_axis_name="c", subcore_axis_name="s")

    @functools.partial(
        pl.kernel,
        mesh=mesh,
        compiler_params=pltpu.CompilerParams(needs_layout_passes=False),
        out_type=jax.ShapeDtypeStruct((B,), jnp.float32),
        scratch_types=[
            pltpu.VMEM((N_COLS, RPW), jnp.float32),
            pltpu.VMEM((N_SPARSE * RPW // 4,), jnp.int32),
            pltpu.VMEM((N_SPARSE * RPW // 4,), jnp.int32),
            pltpu.VMEM((N_SPARSE * RPW // 4,), jnp.int32),
            pltpu.VMEM((N_SPARSE * RPW // 4,), jnp.int32),
            pltpu.VMEM((N_SPARSE * RPW // 4,), jnp.float32),
            pltpu.VMEM((N_SPARSE * RPW // 4,), jnp.float32),
            pltpu.VMEM((N_SPARSE * RPW // 4,), jnp.float32),
            pltpu.VMEM((N_SPARSE * RPW // 4,), jnp.float32),
            pltpu.VMEM((RPW,), jnp.float32),
            pltpu.VMEM((N_DENSE,), jnp.float32),
            pltpu.VMEM((1,), jnp.float32),
            pltpu.SemaphoreType.DMA,
            pltpu.SemaphoreType.DMA,
            pltpu.SemaphoreType.DMA,
            pltpu.SemaphoreType.DMA,
            pltpu.SemaphoreType.DMA,
        ],
    )
    def k(x_hbm, table_hbm, w_hbm, b_hbm, out_hbm,
          x_v, i0, i1, i2, i3, e0, e1, e2, e3, acc_v, w_v, b_v,
          s0, s1, s2, s3, sem_o):
        wid = lax.axis_index("s") * NC + lax.axis_index("c")
        base = wid * RPW
        RQ = RPW // 4
        GQ = NG // 4
        cx1 = pltpu.async_copy(
            x_hbm.at[pl.ds(0, 32), pl.ds(base, RPW)], x_v.at[pl.ds(0, 32), :],
            s0)
        cx2 = pltpu.async_copy(
            x_hbm.at[pl.ds(32, N_COLS - 32), pl.ds(base, RPW)],
            x_v.at[pl.ds(32, N_COLS - 32), :], sem_o)
        cw = pltpu.async_copy(w_hbm, w_v, sem_o)
        cb = pltpu.async_copy(b_hbm, b_v, sem_o)
        lanes = lax.iota(jnp.int32, L)

        quarters = ((0, i0, e0, s0), (1, i1, e1, s1),
                    (2, i2, e2, s2), (3, i3, e3, s3))
        gathers = []
        cx1.wait()
        for q, idx_v, emb_v, sem in quarters:
            for j in range(N_SPARSE):
                for u in range(GQ):
                    xf = x_v[j, pl.ds(q * RQ + u * L, L)]
                    idx_v[pl.ds(j * RQ + u * L, L)] = (
                        xf.astype(jnp.int32) + j * VOCAB
                    )
            gathers.append(
                pltpu.async_copy(table_hbm.at[0].at[idx_v], emb_v, sem)
            )

        cx2.wait()
        cw.wait()
        cb.wait()
        wv = plsc.load_gather(w_v, [jnp.minimum(lanes, N_DENSE - 1)])
        bias_vec = plsc.load_gather(b_v, [jnp.zeros((L,), jnp.int32)])
        for g in range(NG):
            acc = bias_vec
            for d in range(N_DENSE):
                acc = acc + x_v[N_SPARSE + d, pl.ds(g * L, L)] * wv[d]
            acc_v[pl.ds(g * L, L)] = acc

        outs = []
        for q, idx_v, emb_v, sem in quarters:
            gathers[q].wait()
            for u in range(GQ):
                g = q * GQ + u
                s = acc_v[pl.ds(g * L, L)]
                for j in range(N_SPARSE):
                    s = s + emb_v[pl.ds(j * RQ + u * L, L)]
                acc_v[pl.ds(g * L, L)] = s
            outs.append(
                pltpu.async_copy(acc_v.at[pl.ds(q * RQ, RQ)],
                                 out_hbm.at[pl.ds(base + q * RQ, RQ)], sem_o)
            )
        for c in outs:
            c.wait()

    return k


def kernel(X, table, W, bias):
    out = _build()(X.T, table.reshape(1, -1), W.reshape(-1), bias)
    return out.reshape(B, 1)


# speedup vs baseline: 1.0045x; 1.0045x over previous
import functools

import jax
import jax.numpy as jnp
from jax import lax
from jax.experimental import pallas as pl
from jax.experimental.pallas import tpu as pltpu
from jax.experimental.pallas import tpu_sc as plsc

B = 4096
N_SPARSE = 26
N_DENSE = 13
N_COLS = N_SPARSE + N_DENSE
VOCAB = 100000


@functools.cache
def _build():
    info = plsc.get_sparse_core_info()
    NC, NS, L = info.num_cores, info.num_subcores, info.num_lanes
    NW = NC * NS
    RPW = B // NW
    NG = RPW // L
    mesh = plsc.VectorSubcoreMesh(core_axis_name="c", subcore_axis_name="s")

    @functools.partial(
        pl.kernel,
        mesh=mesh,
        compiler_params=pltpu.CompilerParams(needs_layout_passes=False),
        out_type=jax.ShapeDtypeStruct((B,), jnp.float32),
        scratch_types=[
            pltpu.VMEM((N_COLS, RPW), jnp.float32),
            pltpu.VMEM((N_SPARSE * RPW // 2,), jnp.int32),
            pltpu.VMEM((N_SPARSE * RPW // 2,), jnp.int32),
            pltpu.VMEM((N_SPARSE * RPW // 2,), jnp.float32),
            pltpu.VMEM((N_SPARSE * RPW // 2,), jnp.float32),
            pltpu.VMEM((RPW,), jnp.float32),
            pltpu.VMEM((N_DENSE,), jnp.float32),
            pltpu.VMEM((1,), jnp.float32),
            pltpu.SemaphoreType.DMA,
            pltpu.SemaphoreType.DMA,
        ],
    )
    def k(x_hbm, table_hbm, w_hbm, b_hbm, out_hbm,
          x_v, idx_a, idx_b, emb_a, emb_b, acc_v, w_v, b_v, sem_a, sem_b):
        wid = lax.axis_index("s") * NC + lax.axis_index("c")
        base = wid * RPW
        HF = N_SPARSE // 2
        cx1 = pltpu.async_copy(
            x_hbm.at[pl.ds(0, 16), pl.ds(base, RPW)], x_v.at[pl.ds(0, 16), :],
            sem_a)
        cx2 = pltpu.async_copy(
            x_hbm.at[pl.ds(16, 16), pl.ds(base, RPW)], x_v.at[pl.ds(16, 16), :],
            sem_b)
        cx3 = pltpu.async_copy(
            x_hbm.at[pl.ds(32, N_COLS - 32), pl.ds(base, RPW)],
            x_v.at[pl.ds(32, N_COLS - 32), :], sem_b)
        cw = pltpu.async_copy(w_hbm, w_v, sem_b)
        cb = pltpu.async_copy(b_hbm, b_v, sem_b)
        lanes = lax.iota(jnp.int32, L)

        gathers = []
        cx1.wait()
        for h, idx_v, emb_v, sem in ((0, idx_a, emb_a, sem_a),
                                     (1, idx_b, emb_b, sem_b)):
            if h == 1:
                cx2.wait()
                cx3.wait()
                cw.wait()
                cb.wait()
            for jj in range(HF):
                j = h * HF + jj
                for g in range(NG):
                    xf = x_v[j, pl.ds(g * L, L)]
                    idx_v[pl.ds(jj * RPW + g * L, L)] = (
                        xf.astype(jnp.int32) + j * VOCAB
                    )
            gathers.append(
                pltpu.async_copy(table_hbm.at[0].at[idx_v], emb_v, sem)
            )

        wv = plsc.load_gather(w_v, [jnp.minimum(lanes, N_DENSE - 1)])
        bias_vec = plsc.load_gather(b_v, [jnp.zeros((L,), jnp.int32)])
        for g in range(NG):
            acc = bias_vec
            for d in range(N_DENSE):
                acc = acc + x_v[N_SPARSE + d, pl.ds(g * L, L)] * wv[d]
            acc_v[pl.ds(g * L, L)] = acc

        for h, emb_v in ((0, emb_a), (1, emb_b)):
            gathers[h].wait()
            for g in range(NG):
                s = acc_v[pl.ds(g * L, L)]
                for jj in range(HF):
                    s = s + emb_v[pl.ds(jj * RPW + g * L, L)]
                acc_v[pl.ds(g * L, L)] = s

        pltpu.sync_copy(acc_v, out_hbm.at[pl.ds(base, RPW)])

    return k


def kernel(X, table, W, bias):
    out = _build()(X.T, table.reshape(1, -1), W.reshape(-1), bias)
    return out.reshape(B, 1)
